# baseline (speedup 1.0000x reference)
import functools

import jax
import jax.numpy as jnp
from jax import lax
from jax.experimental import pallas as pl
from jax.experimental.pallas import tpu as pltpu
from jax.experimental.pallas import tpu_sc as plsc

_NC = 2
_NS = 16
_NW = _NC * _NS
_L = 16

_CHUNK = 64


_NB = 4
_W = 256


@functools.partial(jax.jit, static_argnames=("R", "D"))
def _detile(embT, tail, *, R, D):
    n_tc = R // 128
    rem = R % 128
    out_rows = (R * D) // 128
    orpr = _W * D // 128

    n_p = n_tc * 128 // _W
    full_r = n_p // _NW
    tail_w = n_p % _NW
    assert (full_r - 1) % _NB == 0 and rem % 8 == 0
    loop_r = full_r - 1

    mesh = plsc.VectorSubcoreMesh(core_axis_name="c", subcore_axis_name="s")

    @functools.partial(
        pl.kernel,
        out_type=jax.ShapeDtypeStruct((out_rows * 128,), jnp.float32),
        mesh=mesh,
        compiler_params=pltpu.CompilerParams(needs_layout_passes=False),
        scratch_types=(
            [pltpu.VMEM((_L, _W), jnp.float32)] * _NB
            + [pltpu.VMEM((_W * D,), jnp.float32)] * _NB
            + [pltpu.SemaphoreType.DMA] * (2 * _NB)
        ),
    )
    def k1(embT_hbm, tail_hbm, out_hbm, *bufs):
        ins = bufs[:_NB]
        ots = bufs[_NB:2 * _NB]
        sis = bufs[2 * _NB:3 * _NB]
        sos = bufs[3 * _NB:4 * _NB]
        wid = lax.axis_index("s") * _NC + lax.axis_index("c")
        iota16 = lax.iota(jnp.int32, _L)

        def issue_in(r, b):
            col = pl.multiple_of((r * _NW + wid) * _W, _W)
            pltpu.async_copy(embT_hbm.at[pl.ds(0, 8), pl.ds(col, _W)],
                             ins[b].at[pl.ds(0, 8)], sis[b])
            pltpu.async_copy(embT_hbm.at[pl.ds(8, 8), pl.ds(col, _W)],
                             ins[b].at[pl.ds(8, 8)], sis[b])

        def wait_in(b):
            pltpu.make_async_copy(embT_hbm.at[pl.ds(0, 8), pl.ds(0, _W)],
                                  ins[b].at[pl.ds(0, 8)], sis[b]).wait()
            pltpu.make_async_copy(embT_hbm.at[pl.ds(0, 8), pl.ds(0, _W)],
                                  ins[b].at[pl.ds(8, 8)], sis[b]).wait()

        def transpose(b, ncols):
            def tb(j, carry):
                c0 = j * 32
                vecs = []
                for i in range(32):
                    cvec = jnp.broadcast_to(c0 + i, (_L,))
                    vecs.append(plsc.load_gather(ins[b], [iota16, cvec]))
                for i in range(32):
                    pos = (c0 + i) * D + iota16
                    plsc.store_scatter(ots[b], [pos], vecs[i])
                return carry

            lax.fori_loop(0, ncols // 32, tb, 0)

        def issue_out(r, b):
            off = pl.multiple_of((r * _NW + wid) * (_W * D), _W * D)
            pltpu.async_copy(ots[b], out_hbm.at[pl.ds(off, _W * D)], sos[b])

        def wait_out(b):
            pltpu.make_async_copy(out_hbm.at[pl.ds(0, _W * D)],
                                  ots[b], sos[b]).wait()

        for r0 in range(_NB - 1):
            issue_in(r0, r0)

        def body(g, carry):
            for b in range(_NB):
                r = g * _NB + b
                wait_in(b)

                nxt = r + (_NB - 1)

                @pl.when(nxt < loop_r)
                def _():
                    issue_in(nxt, (b + _NB - 1) % _NB)

                @pl.when(r >= _NB)
                def _():
                    wait_out(b)

                transpose(b, _W)
                issue_out(r, b)
            return carry

        lax.fori_loop(0, loop_r // _NB, body, 0)

        lb = loop_r % _NB
        issue_in(loop_r, lb)
        wait_in(lb)
        wait_out(lb)
        transpose(lb, _W)
        issue_out(loop_r, lb)

        for b in range(_NB):
            wait_out(b) if b != lb else None
        wait_out(lb)

        @pl.when(wid < tail_w)
        def _():
            col = pl.multiple_of((full_r * _NW + wid) * _W, _W)
            pltpu.sync_copy(embT_hbm.at[pl.ds(0, 8), pl.ds(col, _W)],
                            ins[0].at[pl.ds(0, 8)])
            pltpu.sync_copy(embT_hbm.at[pl.ds(8, 8), pl.ds(col, _W)],
                            ins[0].at[pl.ds(8, 8)])
            transpose(0, _W)
            off = pl.multiple_of((full_r * _NW + wid) * (_W * D), _W * D)
            pltpu.sync_copy(ots[0], out_hbm.at[pl.ds(off, _W * D)])

        if rem:
            @pl.when(wid == _NW - 1)
            def _():
                pltpu.sync_copy(tail_hbm, ots[0].at[pl.ds(0, rem * D)])
                pltpu.sync_copy(ots[0].at[pl.ds(0, rem * D)],
                                out_hbm.at[pl.ds(n_tc * 128 * D, rem * D)])

    return k1(embT, tail)


@functools.partial(jax.jit, static_argnames=("B", "F", "V", "D"))
def _fm_sc(cat_flat, emb, lr_flat, bias, *, B, F, V, D):
    rows_per_w = B // _NW
    n_chunks = rows_per_w // _CHUNK
    idx_len = _CHUNK * F

    mesh = plsc.VectorSubcoreMesh(core_axis_name="c", subcore_axis_name="s")

    @functools.partial(
        pl.kernel,
        out_type=jax.ShapeDtypeStruct((B,), jnp.float32),
        mesh=mesh,
        compiler_params=pltpu.CompilerParams(needs_layout_passes=False,
                                             use_tc_tiling_on_sc=False),
        scratch_types=[
            pltpu.VMEM((idx_len,), jnp.int32),
            pltpu.VMEM((F, _CHUNK), jnp.int32),
            pltpu.VMEM((F * _CHUNK, D), jnp.float32),
            pltpu.VMEM((F, _CHUNK), jnp.float32),
            pltpu.VMEM((B // _NW,), jnp.float32),
            pltpu.SemaphoreType.DMA,
        ],
    )
    def fm_kernel(cat_hbm, emb_hbm, lr_hbm, out_hbm,
                  idxraw_v, idx_v, ebuf, lbuf, out_v, sem):
        wid = lax.axis_index("s") * _NC + lax.axis_index("c")
        w_base = wid * (rows_per_w * F)

        zeros16 = jnp.zeros((_L,), jnp.float32)

        jlane = lax.iota(jnp.int32, _L)
        jF = jlane * F

        def chunk_body(c, carry):
            src_off = pl.multiple_of(w_base + c * idx_len, idx_len)
            pltpu.sync_copy(cat_hbm.at[pl.ds(src_off, idx_len)], idxraw_v)

            for f in range(F):
                for g in range(_CHUNK // _L):
                    addr = jF + (g * _L * F + f)
                    vals = plsc.load_gather(idxraw_v, [addr])
                    idx_v[f, pl.ds(g * _L, _L)] = vals + (f * V)

            copies = []
            for f in range(F):
                copies.append(pltpu.async_copy(
                    emb_hbm.at[idx_v.at[f]],
                    ebuf.at[pl.ds(f * _CHUNK, _CHUNK)], sem))
                copies.append(pltpu.async_copy(
                    lr_hbm.at[idx_v.at[f]], lbuf.at[f], sem))
            for cp in copies:
                cp.wait()

            for g in range(_CHUNK // _L):
                jrow = jlane + (g * _L)
                rowv = [jrow + f * _CHUNK for f in range(F)]

                def d_body(d, acc):
                    ss, q = acc
                    dcol = jnp.broadcast_to(d, (_L,))
                    t = zeros16
                    for f in range(F):
                        e = plsc.load_gather(ebuf, [rowv[f], dcol])
                        t = t + e
                        q = q + e * e
                    return ss + t * t, q

                ss, q = lax.fori_loop(0, D, d_body, (zeros16, zeros16))

                fo = zeros16
                for f in range(F):
                    fo = fo + lbuf[f, pl.ds(g * _L, _L)]

                res = 0.5 * (ss - q) + fo
                dst = pl.multiple_of(c * _CHUNK + g * _L, _L)
                out_v[pl.ds(dst, _L)] = res
            return carry

        lax.fori_loop(0, n_chunks, chunk_body, 0)

        out_off = pl.multiple_of(wid * rows_per_w, rows_per_w)
        pltpu.sync_copy(out_v, out_hbm.at[pl.ds(out_off, rows_per_w)])

    return fm_kernel(cat_flat, emb, lr_flat) + bias


def kernel(cat_indices, embed_table, lr_weight, lr_bias):
    B, F = cat_indices.shape
    D = embed_table.shape[1]
    V = embed_table.shape[0] // F
    R = F * V
    assert B % (_NW * _CHUNK) == 0 and D == _L
    assert R % 8 == 0 and (R % 128) in (0, 64)

    rem = R % 128
    tail = embed_table[R - rem:, :].reshape(rem * D)
    tab = _detile(embed_table.T, tail, R=R, D=D).reshape(R, D)
    cat_flat = cat_indices.astype(jnp.int32).reshape(B * F)
    lr_flat = lr_weight.reshape(-1)
    out = _fm_sc(cat_flat, tab, lr_flat, lr_bias, B=B, F=F, V=V, D=D)
    return out[:, None]


# speedup vs baseline: 1.4829x; 1.4829x over previous
import functools

import jax
import jax.numpy as jnp
from jax import lax
from jax.experimental import pallas as pl
from jax.experimental.pallas import tpu as pltpu
from jax.experimental.pallas import tpu_sc as plsc

_NC = 2
_NS = 16
_NW = _NC * _NS
_L = 16

_CHUNK = 64


_NB = 4
_W = 256


@functools.partial(jax.jit, static_argnames=("R", "D"))
def _detile(embT, tail, *, R, D):
    n_tc = R // 128
    rem = R % 128
    out_rows = (R * D) // 128
    orpr = _W * D // 128

    n_p = n_tc * 128 // _W
    full_r = n_p // _NW
    tail_w = n_p % _NW
    assert (full_r - 1) % _NB == 0 and rem % 8 == 0
    loop_r = full_r - 1

    mesh = plsc.VectorSubcoreMesh(core_axis_name="c", subcore_axis_name="s")

    @functools.partial(
        pl.kernel,
        out_type=jax.ShapeDtypeStruct((out_rows * 128,), jnp.float32),
        mesh=mesh,
        compiler_params=pltpu.CompilerParams(needs_layout_passes=False),
        scratch_types=(
            [pltpu.VMEM((_L, _W), jnp.float32)] * _NB
            + [pltpu.VMEM((_W * D,), jnp.float32)] * _NB
            + [pltpu.SemaphoreType.DMA] * (2 * _NB)
        ),
    )
    def k1(embT_hbm, tail_hbm, out_hbm, *bufs):
        ins = bufs[:_NB]
        ots = bufs[_NB:2 * _NB]
        sis = bufs[2 * _NB:3 * _NB]
        sos = bufs[3 * _NB:4 * _NB]
        wid = lax.axis_index("s") * _NC + lax.axis_index("c")
        iota16 = lax.iota(jnp.int32, _L)

        def issue_in(r, b):
            col = pl.multiple_of((r * _NW + wid) * _W, _W)
            pltpu.async_copy(embT_hbm.at[pl.ds(0, 8), pl.ds(col, _W)],
                             ins[b].at[pl.ds(0, 8)], sis[b])
            pltpu.async_copy(embT_hbm.at[pl.ds(8, 8), pl.ds(col, _W)],
                             ins[b].at[pl.ds(8, 8)], sis[b])

        def wait_in(b):
            pltpu.make_async_copy(embT_hbm.at[pl.ds(0, 8), pl.ds(0, _W)],
                                  ins[b].at[pl.ds(0, 8)], sis[b]).wait()
            pltpu.make_async_copy(embT_hbm.at[pl.ds(0, 8), pl.ds(0, _W)],
                                  ins[b].at[pl.ds(8, 8)], sis[b]).wait()

        dstride = iota16 * D

        def transpose(b, ncols):
            for c0 in range(0, ncols, _L):
                vecs = [ins[b][d, pl.ds(c0, _L)] for d in range(D)]
                for d in range(D):
                    plsc.store_scatter(
                        ots[b], [dstride + (c0 * D + d)], vecs[d])

        def issue_out(r, b):
            off = pl.multiple_of((r * _NW + wid) * (_W * D), _W * D)
            pltpu.async_copy(ots[b], out_hbm.at[pl.ds(off, _W * D)], sos[b])

        def wait_out(b):
            pltpu.make_async_copy(out_hbm.at[pl.ds(0, _W * D)],
                                  ots[b], sos[b]).wait()

        for r0 in range(_NB - 1):
            issue_in(r0, r0)

        def body(g, carry):
            for b in range(_NB):
                r = g * _NB + b
                wait_in(b)

                nxt = r + (_NB - 1)

                @pl.when(nxt < loop_r)
                def _():
                    issue_in(nxt, (b + _NB - 1) % _NB)

                @pl.when(r >= _NB)
                def _():
                    wait_out(b)

                transpose(b, _W)
                issue_out(r, b)
            return carry

        lax.fori_loop(0, loop_r // _NB, body, 0)

        lb = loop_r % _NB
        issue_in(loop_r, lb)
        wait_in(lb)
        wait_out(lb)
        transpose(lb, _W)
        issue_out(loop_r, lb)

        for b in range(_NB):
            wait_out(b) if b != lb else None
        wait_out(lb)

        @pl.when(wid < tail_w)
        def _():
            col = pl.multiple_of((full_r * _NW + wid) * _W, _W)
            pltpu.sync_copy(embT_hbm.at[pl.ds(0, 8), pl.ds(col, _W)],
                            ins[0].at[pl.ds(0, 8)])
            pltpu.sync_copy(embT_hbm.at[pl.ds(8, 8), pl.ds(col, _W)],
                            ins[0].at[pl.ds(8, 8)])
            transpose(0, _W)
            off = pl.multiple_of((full_r * _NW + wid) * (_W * D), _W * D)
            pltpu.sync_copy(ots[0], out_hbm.at[pl.ds(off, _W * D)])

        if rem:
            @pl.when(wid == _NW - 1)
            def _():
                pltpu.sync_copy(tail_hbm, ots[0].at[pl.ds(0, rem * D)])
                pltpu.sync_copy(ots[0].at[pl.ds(0, rem * D)],
                                out_hbm.at[pl.ds(n_tc * 128 * D, rem * D)])

    return k1(embT, tail)


@functools.partial(jax.jit, static_argnames=("B", "F", "V", "D"))
def _fm_sc(cat_flat, emb, lr_flat, bias, *, B, F, V, D):
    rows_per_w = B // _NW
    n_chunks = rows_per_w // _CHUNK
    idx_len = _CHUNK * F

    mesh = plsc.VectorSubcoreMesh(core_axis_name="c", subcore_axis_name="s")

    @functools.partial(
        pl.kernel,
        out_type=jax.ShapeDtypeStruct((B,), jnp.float32),
        mesh=mesh,
        compiler_params=pltpu.CompilerParams(needs_layout_passes=False,
                                             use_tc_tiling_on_sc=False),
        scratch_types=[
            pltpu.VMEM((idx_len,), jnp.int32),
            pltpu.VMEM((F, _CHUNK), jnp.int32),
            pltpu.VMEM((F * _CHUNK, D), jnp.float32),
            pltpu.VMEM((F, _CHUNK), jnp.float32),
            pltpu.VMEM((B // _NW,), jnp.float32),
            pltpu.SemaphoreType.DMA,
        ],
    )
    def fm_kernel(cat_hbm, emb_hbm, lr_hbm, out_hbm,
                  idxraw_v, idx_v, ebuf, lbuf, out_v, sem):
        wid = lax.axis_index("s") * _NC + lax.axis_index("c")
        w_base = wid * (rows_per_w * F)

        zeros16 = jnp.zeros((_L,), jnp.float32)

        jlane = lax.iota(jnp.int32, _L)
        jF = jlane * F

        def chunk_body(c, carry):
            src_off = pl.multiple_of(w_base + c * idx_len, idx_len)
            pltpu.sync_copy(cat_hbm.at[pl.ds(src_off, idx_len)], idxraw_v)

            for f in range(F):
                for g in range(_CHUNK // _L):
                    addr = jF + (g * _L * F + f)
                    vals = plsc.load_gather(idxraw_v, [addr])
                    idx_v[f, pl.ds(g * _L, _L)] = vals + (f * V)

            copies = []
            for f in range(F):
                copies.append(pltpu.async_copy(
                    emb_hbm.at[idx_v.at[f]],
                    ebuf.at[pl.ds(f * _CHUNK, _CHUNK)], sem))
                copies.append(pltpu.async_copy(
                    lr_hbm.at[idx_v.at[f]], lbuf.at[f], sem))
            for cp in copies:
                cp.wait()

            for g in range(_CHUNK // _L):
                jrow = jlane + (g * _L)
                rowv = [jrow + f * _CHUNK for f in range(F)]

                def d_body(d, acc):
                    ss, q = acc
                    dcol = jnp.broadcast_to(d, (_L,))
                    t = zeros16
                    for f in range(F):
                        e = plsc.load_gather(ebuf, [rowv[f], dcol])
                        t = t + e
                        q = q + e * e
                    return ss + t * t, q

                ss, q = lax.fori_loop(0, D, d_body, (zeros16, zeros16))

                fo = zeros16
                for f in range(F):
                    fo = fo + lbuf[f, pl.ds(g * _L, _L)]

                res = 0.5 * (ss - q) + fo
                dst = pl.multiple_of(c * _CHUNK + g * _L, _L)
                out_v[pl.ds(dst, _L)] = res
            return carry

        lax.fori_loop(0, n_chunks, chunk_body, 0)

        out_off = pl.multiple_of(wid * rows_per_w, rows_per_w)
        pltpu.sync_copy(out_v, out_hbm.at[pl.ds(out_off, rows_per_w)])

    return fm_kernel(cat_flat, emb, lr_flat) + bias


def kernel(cat_indices, embed_table, lr_weight, lr_bias):
    B, F = cat_indices.shape
    D = embed_table.shape[1]
    V = embed_table.shape[0] // F
    R = F * V
    assert B % (_NW * _CHUNK) == 0 and D == _L
    assert R % 8 == 0 and (R % 128) in (0, 64)

    rem = R % 128
    tail = embed_table[R - rem:, :].reshape(rem * D)
    tab = _detile(embed_table.T, tail, R=R, D=D).reshape(R, D)
    cat_flat = cat_indices.astype(jnp.int32).reshape(B * F)
    lr_flat = lr_weight.reshape(-1)
    out = _fm_sc(cat_flat, tab, lr_flat, lr_bias, B=B, F=F, V=V, D=D)
    return out[:, None]


# speedup vs baseline: 2.8490x; 1.9213x over previous
import functools

import jax
import jax.numpy as jnp
from jax import lax
from jax.experimental import pallas as pl
from jax.experimental.pallas import tpu as pltpu
from jax.experimental.pallas import tpu_sc as plsc

_NC = 2
_NS = 16
_NW = _NC * _NS
_L = 16

_CHUNK = 64


_NB = 4
_W = 256


@functools.partial(jax.jit, static_argnames=("R", "D"))
def _detile(embT, tail, *, R, D):
    n_tc = R // 128
    rem = R % 128
    out_rows = (R * D) // 128
    orpr = _W * D // 128

    n_p = n_tc * 128 // _W
    full_r = n_p // _NW
    tail_w = n_p % _NW
    assert (full_r - 1) % _NB == 0 and rem % 8 == 0
    loop_r = full_r - 1

    mesh = plsc.VectorSubcoreMesh(core_axis_name="c", subcore_axis_name="s")

    @functools.partial(
        pl.kernel,
        out_type=jax.ShapeDtypeStruct((out_rows * 128,), jnp.float32),
        mesh=mesh,
        compiler_params=pltpu.CompilerParams(needs_layout_passes=False),
        scratch_types=(
            [pltpu.VMEM((_L, _W), jnp.float32)] * _NB
            + [pltpu.VMEM((_W * D,), jnp.float32)] * _NB
            + [pltpu.SemaphoreType.DMA] * (2 * _NB)
        ),
    )
    def k1(embT_hbm, tail_hbm, out_hbm, *bufs):
        ins = bufs[:_NB]
        ots = bufs[_NB:2 * _NB]
        sis = bufs[2 * _NB:3 * _NB]
        sos = bufs[3 * _NB:4 * _NB]
        wid = lax.axis_index("s") * _NC + lax.axis_index("c")
        iota16 = lax.iota(jnp.int32, _L)

        def issue_in(r, b):
            col = pl.multiple_of((r * _NW + wid) * _W, _W)
            pltpu.async_copy(embT_hbm.at[pl.ds(0, 8), pl.ds(col, _W)],
                             ins[b].at[pl.ds(0, 8)], sis[b])
            pltpu.async_copy(embT_hbm.at[pl.ds(8, 8), pl.ds(col, _W)],
                             ins[b].at[pl.ds(8, 8)], sis[b])

        def wait_in(b):
            pltpu.make_async_copy(embT_hbm.at[pl.ds(0, 8), pl.ds(0, _W)],
                                  ins[b].at[pl.ds(0, 8)], sis[b]).wait()
            pltpu.make_async_copy(embT_hbm.at[pl.ds(0, 8), pl.ds(0, _W)],
                                  ins[b].at[pl.ds(8, 8)], sis[b]).wait()

        dstride = iota16 * D

        def transpose(b, ncols):
            @plsc.parallel_loop(0, ncols, step=_L, unroll=4)
            def _(c0):
                c0 = pl.multiple_of(c0, _L)
                vecs = [ins[b][d, pl.ds(c0, _L)] for d in range(D)]
                for d in range(D):
                    plsc.store_scatter(
                        ots[b], [dstride + (c0 * D + d)], vecs[d])

        def issue_out(r, b):
            off = pl.multiple_of((r * _NW + wid) * (_W * D), _W * D)
            pltpu.async_copy(ots[b], out_hbm.at[pl.ds(off, _W * D)], sos[b])

        def wait_out(b):
            pltpu.make_async_copy(out_hbm.at[pl.ds(0, _W * D)],
                                  ots[b], sos[b]).wait()

        for r0 in range(_NB - 1):
            issue_in(r0, r0)

        def body(g, carry):
            for b in range(_NB):
                r = g * _NB + b
                wait_in(b)

                nxt = r + (_NB - 1)

                @pl.when(nxt < loop_r)
                def _():
                    issue_in(nxt, (b + _NB - 1) % _NB)

                @pl.when(r >= _NB)
                def _():
                    wait_out(b)

                transpose(b, _W)
                issue_out(r, b)
            return carry

        lax.fori_loop(0, loop_r // _NB, body, 0)

        lb = loop_r % _NB
        issue_in(loop_r, lb)
        wait_in(lb)
        wait_out(lb)
        transpose(lb, _W)
        issue_out(loop_r, lb)

        for b in range(_NB):
            wait_out(b) if b != lb else None
        wait_out(lb)

        @pl.when(wid < tail_w)
        def _():
            col = pl.multiple_of((full_r * _NW + wid) * _W, _W)
            pltpu.sync_copy(embT_hbm.at[pl.ds(0, 8), pl.ds(col, _W)],
                            ins[0].at[pl.ds(0, 8)])
            pltpu.sync_copy(embT_hbm.at[pl.ds(8, 8), pl.ds(col, _W)],
                            ins[0].at[pl.ds(8, 8)])
            transpose(0, _W)
            off = pl.multiple_of((full_r * _NW + wid) * (_W * D), _W * D)
            pltpu.sync_copy(ots[0], out_hbm.at[pl.ds(off, _W * D)])

        if rem:
            @pl.when(wid == _NW - 1)
            def _():
                pltpu.sync_copy(tail_hbm, ots[0].at[pl.ds(0, rem * D)])
                pltpu.sync_copy(ots[0].at[pl.ds(0, rem * D)],
                                out_hbm.at[pl.ds(n_tc * 128 * D, rem * D)])

    return k1(embT, tail)


@functools.partial(jax.jit, static_argnames=("B", "F", "V", "D"))
def _fm_sc(cat_flat, emb, lr_flat, bias, *, B, F, V, D):
    rows_per_w = B // _NW
    n_chunks = rows_per_w // _CHUNK
    idx_len = _CHUNK * F

    mesh = plsc.VectorSubcoreMesh(core_axis_name="c", subcore_axis_name="s")

    @functools.partial(
        pl.kernel,
        out_type=jax.ShapeDtypeStruct((B,), jnp.float32),
        mesh=mesh,
        compiler_params=pltpu.CompilerParams(needs_layout_passes=False,
                                             use_tc_tiling_on_sc=False),
        scratch_types=[
            pltpu.VMEM((idx_len,), jnp.int32),
            pltpu.VMEM((F, _CHUNK), jnp.int32),
            pltpu.VMEM((F * _CHUNK, D), jnp.float32),
            pltpu.VMEM((F, _CHUNK), jnp.float32),
            pltpu.VMEM((B // _NW,), jnp.float32),
            pltpu.SemaphoreType.DMA,
        ],
    )
    def fm_kernel(cat_hbm, emb_hbm, lr_hbm, out_hbm,
                  idxraw_v, idx_v, ebuf, lbuf, out_v, sem):
        wid = lax.axis_index("s") * _NC + lax.axis_index("c")
        w_base = wid * (rows_per_w * F)

        zeros16 = jnp.zeros((_L,), jnp.float32)

        jlane = lax.iota(jnp.int32, _L)
        jF = jlane * F

        def chunk_body(c, carry):
            src_off = pl.multiple_of(w_base + c * idx_len, idx_len)
            pltpu.sync_copy(cat_hbm.at[pl.ds(src_off, idx_len)], idxraw_v)

            for f in range(F):
                for g in range(_CHUNK // _L):
                    addr = jF + (g * _L * F + f)
                    vals = plsc.load_gather(idxraw_v, [addr])
                    idx_v[f, pl.ds(g * _L, _L)] = vals + (f * V)

            copies = []
            for f in range(F):
                copies.append(pltpu.async_copy(
                    emb_hbm.at[idx_v.at[f]],
                    ebuf.at[pl.ds(f * _CHUNK, _CHUNK)], sem))
                copies.append(pltpu.async_copy(
                    lr_hbm.at[idx_v.at[f]], lbuf.at[f], sem))
            for cp in copies:
                cp.wait()

            for g in range(_CHUNK // _L):
                jrow = jlane + (g * _L)
                rowv = [jrow + f * _CHUNK for f in range(F)]

                def d_body(d, acc):
                    ss, q = acc
                    dcol = jnp.broadcast_to(d, (_L,))
                    t = zeros16
                    for f in range(F):
                        e = plsc.load_gather(ebuf, [rowv[f], dcol])
                        t = t + e
                        q = q + e * e
                    return ss + t * t, q

                ss, q = lax.fori_loop(0, D, d_body, (zeros16, zeros16))

                fo = zeros16
                for f in range(F):
                    fo = fo + lbuf[f, pl.ds(g * _L, _L)]

                res = 0.5 * (ss - q) + fo
                dst = pl.multiple_of(c * _CHUNK + g * _L, _L)
                out_v[pl.ds(dst, _L)] = res
            return carry

        lax.fori_loop(0, n_chunks, chunk_body, 0)

        out_off = pl.multiple_of(wid * rows_per_w, rows_per_w)
        pltpu.sync_copy(out_v, out_hbm.at[pl.ds(out_off, rows_per_w)])

    return fm_kernel(cat_flat, emb, lr_flat) + bias


def kernel(cat_indices, embed_table, lr_weight, lr_bias):
    B, F = cat_indices.shape
    D = embed_table.shape[1]
    V = embed_table.shape[0] // F
    R = F * V
    assert B % (_NW * _CHUNK) == 0 and D == _L
    assert R % 8 == 0 and (R % 128) in (0, 64)

    rem = R % 128
    tail = embed_table[R - rem:, :].reshape(rem * D)
    tab = _detile(embed_table.T, tail, R=R, D=D).reshape(R, D)
    cat_flat = cat_indices.astype(jnp.int32).reshape(B * F)
    lr_flat = lr_weight.reshape(-1)
    out = _fm_sc(cat_flat, tab, lr_flat, lr_bias, B=B, F=F, V=V, D=D)
    return out[:, None]
